# baseline (speedup 1.0000x reference)
import functools

import jax
import jax.numpy as jnp
from jax import lax
from jax.experimental import pallas as pl
from jax.experimental.pallas import tpu as pltpu
from jax.experimental.pallas import tpu_sc as plsc

_CHUNK = 800


def _prep_sc(table, x, W):
    V, D = table.shape
    B = x.shape[0]
    info = plsc.get_sparse_core_info()
    NC, NS, L = info.num_cores, info.num_subcores, info.num_lanes
    NW = NC * NS
    b_per_w = B // NW
    nchunks = V // _CHUNK
    csteps = _CHUNK // L
    mesh = plsc.VectorSubcoreMesh(core_axis_name="c", subcore_axis_name="s")

    @functools.partial(
        pl.kernel,
        mesh=mesh,
        out_type=(
            jax.ShapeDtypeStruct((B, D), jnp.float32),
            jax.ShapeDtypeStruct((D, V), jnp.float32),
        ),
        scratch_types=[
            pltpu.VMEM((b_per_w,), jnp.int32),
            pltpu.VMEM((b_per_w, D), jnp.float32),
            pltpu.VMEM((_CHUNK, D), jnp.float32),
            pltpu.VMEM((D, _CHUNK), jnp.float32),
            pltpu.SemaphoreType.DMA,
        ],
        compiler_params=pltpu.CompilerParams(
            use_tc_tiling_on_sc=False, needs_layout_passes=False
        ),
    )
    def prep_kernel(table_hbm, idx_hbm, w_hbm, emb_hbm, wt_hbm,
                    idx_v, rows_v, inbuf, outbuf, sem):
        wid = lax.axis_index("s") * NC + lax.axis_index("c")

        base = wid * b_per_w
        pltpu.sync_copy(idx_hbm.at[pl.ds(base, b_per_w)], idx_v)
        pltpu.async_copy(table_hbm.at[idx_v], rows_v, sem).wait()
        pltpu.sync_copy(rows_v, emb_hbm.at[pl.ds(base, b_per_w)])

        iota = lax.iota(jnp.int32, L)

        def do_chunk(chunk):
            c0 = chunk * _CHUNK
            pltpu.sync_copy(w_hbm.at[pl.ds(c0, _CHUNK), :], inbuf)

            def body(c, carry):
                rows = c * L + iota
                zeros = iota * 0
                for h in range(D):
                    v = plsc.load_gather(inbuf, [rows, zeros + h])
                    outbuf[h, pl.ds(c * L, L)] = v
                return carry

            lax.fori_loop(0, csteps, body, jnp.int32(0))
            pltpu.sync_copy(outbuf, wt_hbm.at[:, pl.ds(c0, _CHUNK)])

        for k in range(pl.cdiv(nchunks, NW)):
            chunk = wid + k * NW

            @pl.when(chunk < nchunks)
            def _():
                do_chunk(chunk)

    return prep_kernel(table, x, W)


_TV = 4096


def _project_tc(emb, Wt, b2d):
    B, H = emb.shape
    V = Wt.shape[1]
    nv = pl.cdiv(V, _TV)

    def mm_kernel(emb_ref, wt_ref, b_ref, out_ref):
        out_ref[...] = (
            jnp.dot(emb_ref[...], wt_ref[...], preferred_element_type=jnp.float32)
            + b_ref[...]
        )

    return pl.pallas_call(
        mm_kernel,
        grid=(nv,),
        in_specs=[
            pl.BlockSpec((B, H), lambda i: (0, 0)),
            pl.BlockSpec((H, _TV), lambda i: (0, i)),
            pl.BlockSpec((1, _TV), lambda i: (0, i)),
        ],
        out_specs=pl.BlockSpec((B, _TV), lambda i: (0, i)),
        out_shape=jax.ShapeDtypeStruct((B, V), jnp.float32),
    )(emb, Wt, b2d)


def kernel(x, table, W, b):
    V, H = W.shape
    emb, Wt = _prep_sc(table, x, W)
    return _project_tc(emb, Wt, b.reshape(1, V))


# speedup vs baseline: 1.0479x; 1.0479x over previous
import functools

import jax
import jax.numpy as jnp
from jax import lax
from jax.experimental import pallas as pl
from jax.experimental.pallas import tpu as pltpu
from jax.experimental.pallas import tpu_sc as plsc


def _gather_rows_sc(table, x):
    V, D = table.shape
    B = x.shape[0]
    info = plsc.get_sparse_core_info()
    NC, NS = info.num_cores, info.num_subcores
    NW = NC * NS
    b_per_w = B // NW
    mesh = plsc.VectorSubcoreMesh(core_axis_name="c", subcore_axis_name="s")

    @functools.partial(
        pl.kernel,
        mesh=mesh,
        out_type=jax.ShapeDtypeStruct((B, D), jnp.float32),
        scratch_types=[
            pltpu.VMEM((b_per_w,), jnp.int32),
            pltpu.VMEM((b_per_w, D), jnp.float32),
            pltpu.SemaphoreType.DMA,
        ],
        compiler_params=pltpu.CompilerParams(use_tc_tiling_on_sc=False),
    )
    def gather_kernel(table_hbm, idx_hbm, out_hbm, idx_v, rows_v, sem):
        wid = lax.axis_index("s") * NC + lax.axis_index("c")
        base = wid * b_per_w
        pltpu.sync_copy(idx_hbm.at[pl.ds(base, b_per_w)], idx_v)
        pltpu.async_copy(table_hbm.at[idx_v], rows_v, sem).wait()
        pltpu.sync_copy(rows_v, out_hbm.at[pl.ds(base, b_per_w)])

    return gather_kernel(table, x)


_TV = 4096


def _project_tc(emb, Whi, Wlo, b2d):
    B, H = emb.shape
    V = Whi.shape[0]
    nv = pl.cdiv(V, _TV)

    def mm_kernel(emb_ref, whi_ref, wlo_ref, b_ref, out_ref):
        e = emb_ref[...]
        ehi = e.astype(jnp.bfloat16)
        elo = (e - ehi.astype(jnp.float32)).astype(jnp.bfloat16)
        dims = (((1,), (1,)), ((), ()))

        def dot(a, w):
            return lax.dot_general(
                a, w[...], dims, preferred_element_type=jnp.float32
            )

        out_ref[...] = (
            dot(ehi, whi_ref)
            + dot(ehi, wlo_ref)
            + dot(elo, whi_ref)
            + b_ref[...]
        )

    return pl.pallas_call(
        mm_kernel,
        grid=(nv,),
        in_specs=[
            pl.BlockSpec((B, H), lambda i: (0, 0)),
            pl.BlockSpec((_TV, H), lambda i: (i, 0)),
            pl.BlockSpec((_TV, H), lambda i: (i, 0)),
            pl.BlockSpec((1, _TV), lambda i: (0, i)),
        ],
        out_specs=pl.BlockSpec((B, _TV), lambda i: (0, i)),
        out_shape=jax.ShapeDtypeStruct((B, V), jnp.float32),
    )(emb, Whi, Wlo, b2d)


def kernel(x, table, W, b):
    V, H = W.shape
    emb = _gather_rows_sc(table, x)
    Whi = W.astype(jnp.bfloat16)
    Wlo = (W - Whi.astype(jnp.float32)).astype(jnp.bfloat16)
    return _project_tc(emb, Whi, Wlo, b.reshape(1, V))


# speedup vs baseline: 1.1549x; 1.1021x over previous
import functools

import jax
import jax.numpy as jnp
from jax import lax
from jax.experimental import pallas as pl
from jax.experimental.pallas import tpu as pltpu
from jax.experimental.pallas import tpu_sc as plsc


def _gather_rows_sc(table, x):
    V, D = table.shape
    B = x.shape[0]
    info = plsc.get_sparse_core_info()
    NC, NS = info.num_cores, info.num_subcores
    NW = NC * NS
    b_per_w = B // NW
    mesh = plsc.VectorSubcoreMesh(core_axis_name="c", subcore_axis_name="s")

    @functools.partial(
        pl.kernel,
        mesh=mesh,
        out_type=jax.ShapeDtypeStruct((B, D), jnp.float32),
        scratch_types=[
            pltpu.VMEM((b_per_w,), jnp.int32),
            pltpu.VMEM((b_per_w, D), jnp.float32),
            pltpu.SemaphoreType.DMA,
        ],
        compiler_params=pltpu.CompilerParams(use_tc_tiling_on_sc=False),
    )
    def gather_kernel(table_hbm, idx_hbm, out_hbm, idx_v, rows_v, sem):
        wid = lax.axis_index("s") * NC + lax.axis_index("c")
        base = wid * b_per_w
        pltpu.sync_copy(idx_hbm.at[pl.ds(base, b_per_w)], idx_v)
        pltpu.async_copy(table_hbm.at[idx_v], rows_v, sem).wait()
        pltpu.sync_copy(rows_v, out_hbm.at[pl.ds(base, b_per_w)])

    return gather_kernel(table, x)


_TV = 4096


def _project_tc(emb, Wthi, Wtlo, b2d):
    B, H = emb.shape
    V = Wthi.shape[1]
    nv = pl.cdiv(V, _TV)

    def mm_kernel(emb_ref, whi_ref, wlo_ref, b_ref, out_ref):
        e = emb_ref[...]
        ehi = e.astype(jnp.bfloat16)
        elo = (e - ehi.astype(jnp.float32)).astype(jnp.bfloat16)

        def dot(a, w):
            return jnp.dot(a, w[...], preferred_element_type=jnp.float32)

        out_ref[...] = (
            dot(ehi, whi_ref)
            + dot(ehi, wlo_ref)
            + dot(elo, whi_ref)
            + b_ref[...]
        )

    return pl.pallas_call(
        mm_kernel,
        grid=(nv,),
        in_specs=[
            pl.BlockSpec((B, H), lambda i: (0, 0)),
            pl.BlockSpec((H, _TV), lambda i: (0, i)),
            pl.BlockSpec((H, _TV), lambda i: (0, i)),
            pl.BlockSpec((1, _TV), lambda i: (0, i)),
        ],
        out_specs=pl.BlockSpec((B, _TV), lambda i: (0, i)),
        out_shape=jax.ShapeDtypeStruct((B, V), jnp.float32),
    )(emb, Wthi, Wtlo, b2d)


def kernel(x, table, W, b):
    V, H = W.shape
    emb = _gather_rows_sc(table, x)
    Whi = W.astype(jnp.bfloat16)
    Wlo = (W - Whi.astype(jnp.float32)).astype(jnp.bfloat16)
    return _project_tc(emb, Whi.T, Wlo.T, b.reshape(1, V))


# speedup vs baseline: 1.2614x; 1.0922x over previous
import functools

import jax
import jax.numpy as jnp
from jax import lax
from jax.experimental import pallas as pl
from jax.experimental.pallas import tpu as pltpu
from jax.experimental.pallas import tpu_sc as plsc


def _gather_rows_sc(table, x):
    V, D = table.shape
    B = x.shape[0]
    info = plsc.get_sparse_core_info()
    NC, NS = info.num_cores, info.num_subcores
    NW = NC * NS
    b_per_w = B // NW
    mesh = plsc.VectorSubcoreMesh(core_axis_name="c", subcore_axis_name="s")

    @functools.partial(
        pl.kernel,
        mesh=mesh,
        out_type=jax.ShapeDtypeStruct((B, D), jnp.float32),
        scratch_types=[
            pltpu.VMEM((b_per_w,), jnp.int32),
            pltpu.VMEM((b_per_w, D), jnp.float32),
            pltpu.SemaphoreType.DMA,
        ],
        compiler_params=pltpu.CompilerParams(use_tc_tiling_on_sc=False),
    )
    def gather_kernel(table_hbm, idx_hbm, out_hbm, idx_v, rows_v, sem):
        wid = lax.axis_index("s") * NC + lax.axis_index("c")
        base = wid * b_per_w
        pltpu.sync_copy(idx_hbm.at[pl.ds(base, b_per_w)], idx_v)
        pltpu.async_copy(table_hbm.at[idx_v], rows_v, sem).wait()
        pltpu.sync_copy(rows_v, out_hbm.at[pl.ds(base, b_per_w)])

    return gather_kernel(table, x)


_TV = 4096


def _project_tc(emb, Wt, b2d):
    B, H = emb.shape
    V = Wt.shape[1]
    nv = pl.cdiv(V, _TV)

    def mm_kernel(emb_ref, wt_ref, b_ref, out_ref):
        out_ref[...] = (
            jnp.dot(emb_ref[...], wt_ref[...], preferred_element_type=jnp.float32)
            + b_ref[...]
        )

    return pl.pallas_call(
        mm_kernel,
        grid=(nv,),
        in_specs=[
            pl.BlockSpec((B, H), lambda i: (0, 0)),
            pl.BlockSpec((H, _TV), lambda i: (0, i)),
            pl.BlockSpec((1, _TV), lambda i: (0, i)),
        ],
        out_specs=pl.BlockSpec((B, _TV), lambda i: (0, i)),
        out_shape=jax.ShapeDtypeStruct((B, V), jnp.float32),
    )(emb, Wt, b2d)


def kernel(x, table, W, b):
    V, H = W.shape
    emb = _gather_rows_sc(table, x)
    return _project_tc(emb, W.T, b.reshape(1, V))
